# baseline (speedup 1.0000x reference)
import functools

import jax
import jax.numpy as jnp
from jax import lax
from jax.experimental import pallas as pl
from jax.experimental.pallas import tpu as pltpu
from jax.experimental.pallas import tpu_sc as plsc

_N = 100000
_E = 3200000
_C = 7
_ROW = 8
_SEMI_LAMBDA = 0.001
_EDGE_LAMBDA = 1.0

_NC = 2
_NS = 16
_NW = _NC * _NS
_EPW = _E // _NW
_B = 800
_NCHUNK = _EPW // _B
_NPAIR = (_NCHUNK - 1) // 2
_SUB = 80
_NSUB = _B // _SUB
_STEPS = _B // 16
_SROWS = 10
_NPT = 3136
_NPAD = _NW * _NPT
_NSTEPS = _NPT // 16
_MROWS = _NPAD // 64

_LN2 = 0.6931471805599453


def _vlog(x):
    bits = lax.bitcast_convert_type(x, jnp.int32)
    e = (bits >> 23) - 127
    m = lax.bitcast_convert_type(
        (bits & jnp.int32(0x007FFFFF)) | jnp.int32(0x3F800000), jnp.float32)
    big = m > jnp.float32(1.41421356)
    m = jnp.where(big, m * jnp.float32(0.5), m)
    e = jnp.where(big, e + 1, e)
    f = m - jnp.float32(1.0)
    z = f * f
    p = jnp.float32(1.4249322787e-1)
    p = p * f + jnp.float32(-1.6668057665e-1)
    p = p * f + jnp.float32(2.0000714765e-1)
    p = p * f + jnp.float32(-2.4999993993e-1)
    p = p * f + jnp.float32(3.3333331174e-1)
    r = f * z * p - jnp.float32(0.5) * z + f
    return r + e.astype(jnp.float32) * jnp.float32(_LN2)


def _body(*refs):
    (pe, edg, pnd, mta, out) = refs[:5]
    (isl0, isl1, sep00, sep01, sep10, sep11, own0, own1,
     rec00, rec01, rec10, rec11,
     metat, npn, accout,
     semi0, semi1, semg0, semg1) = refs[5:]

    wid = lax.axis_index("s") * _NC + lax.axis_index("c")
    la = lax.iota(jnp.int32, 16)
    cols = [jnp.full((16,), c, jnp.int32) for c in range(_ROW)]
    czero = jnp.zeros((16,), jnp.int32)
    cone = jnp.full((16,), 1, jnp.int32)
    zero = jnp.zeros((16,), jnp.float32)
    one = jnp.full((16,), 1.0, jnp.float32)

    def meta_of(i):
        w = plsc.load_gather(metat, [i >> 6, (i >> 2) & 15])
        return (w >> ((i & 3) << 3)) & 15

    ebase = wid * _EPW

    def start_fetch(ci, islab, ownbuf, sem):
        base = ebase + ci * _B
        pltpu.async_copy(edg.at[pl.ds(base, _B)], islab, sem)
        pltpu.async_copy(pe.at[pl.ds(base, _B)], ownbuf, sem)

    def wait_fetch(islab, ownbuf, sem):
        pltpu.make_async_copy(edg.at[pl.ds(0, _B)], islab, sem).wait()
        pltpu.make_async_copy(pe.at[pl.ds(0, _B)], ownbuf, sem).wait()

    def separate(islab, sep0, sep1):
        def srow(r, _):
            for u in range(5):
                j = r * 80 + u * 16 + la
                sep0[r, pl.ds(u * 16, 16)] = plsc.load_gather(
                    islab, [j, czero])
                sep1[r, pl.ds(u * 16, 16)] = plsc.load_gather(
                    islab, [j, cone])
            return 0
        lax.fori_loop(0, _SROWS, srow, 0)

    def issue_gathers(sep0, sep1, r0buf, r1buf, sem):
        for r in range(_NSUB):
            pltpu.async_copy(
                pe.at[sep0.at[r]], r0buf.at[pl.ds(r * _SUB, _SUB)], sem)
            pltpu.async_copy(
                pe.at[sep1.at[r]], r1buf.at[pl.ds(r * _SUB, _SUB)], sem)

    def wait_gathers(r0buf, r1buf, sem):
        pltpu.make_async_copy(pe.at[pl.ds(0, _B)], r0buf, sem).wait()
        pltpu.make_async_copy(pe.at[pl.ds(0, _B)], r1buf, sem).wait()

    def compute(islab, own, rec0, rec1, carry):
        def step(t, c2):
            s2, slog, sden, slast = c2
            j = t * 16 + la
            i0 = plsc.load_gather(islab, [j, czero])
            i1 = plsc.load_gather(islab, [j, cone])
            acc = jnp.zeros((16,), jnp.float32)
            for c in range(_ROW):
                a = plsc.load_gather(rec0, [j, cols[c]])
                b = plsc.load_gather(rec1, [j, cols[c]])
                d = a - b
                acc = acc + d * d
            mt0 = meta_of(i0)
            mt1 = meta_of(i1)
            g0 = mt0 & 7
            g1 = mt1 & 7
            m0 = mt0 >> 3
            m1 = mt1 >> 3
            plast = plsc.load_gather(own, [j, cols[7]])
            p0 = plsc.load_gather(own, [j, g0])
            p1 = plsc.load_gather(own, [j, g1])
            m0b = m0 == 1
            m1b = m1 == 1
            bothb = (m0 & m1) == 1
            anyi = m0 | m1
            sameb = g0 == g1
            arg1 = jnp.where(sameb, p0, plast)
            arg2 = jnp.where(m0b, plast + p0,
                             jnp.where(m1b, plast + p1, one))
            arg = jnp.where(bothb, arg1, arg2)
            return (s2 + acc, slog + _vlog(arg), sden + anyi,
                    slast + plast)

        return lax.fori_loop(0, _STEPS, step, carry)

    start_fetch(0, isl0, own0, semi0)

    pltpu.sync_copy(mta, metat)
    nbase = wid * _NPT
    pltpu.sync_copy(pnd.at[pl.ds(nbase, _NPT)], npn)

    def nstep(t, carry):
        nlog, ncnt = carry
        j = t * 16 + la
        mt = meta_of(nbase + j)
        g = mt & 7
        mv = (mt >> 3).astype(jnp.float32)
        pn = plsc.load_gather(npn, [j, g])
        return nlog + mv * _vlog(pn), ncnt + mv

    nlog, ncnt = lax.fori_loop(0, _NSTEPS, nstep, (zero, zero))

    wait_fetch(isl0, own0, semi0)
    separate(isl0, sep00, sep01)
    issue_gathers(sep00, sep01, rec00, rec01, semg0)
    start_fetch(1, isl1, own1, semi1)

    izero = jnp.zeros((16,), jnp.int32)

    def pair(k, carry):
        c0 = 2 * k
        s2, slog, sden, slast = carry
        wait_fetch(isl1, own1, semi1)
        separate(isl1, sep10, sep11)
        issue_gathers(sep10, sep11, rec10, rec11, semg1)
        wait_gathers(rec00, rec01, semg0)
        carry = compute(isl0, own0, rec00, rec01, (s2, slog, sden, slast))
        start_fetch(c0 + 2, isl0, own0, semi0)
        wait_fetch(isl0, own0, semi0)
        separate(isl0, sep00, sep01)
        issue_gathers(sep00, sep01, rec00, rec01, semg0)
        wait_gathers(rec10, rec11, semg1)
        carry = compute(isl1, own1, rec10, rec11, carry)

        @pl.when(k < _NPAIR - 1)
        def _():
            start_fetch(c0 + 3, isl1, own1, semi1)
        return carry

    sdeni = izero
    carry = lax.fori_loop(0, _NPAIR, pair, (zero, zero, sdeni, zero))

    wait_gathers(rec00, rec01, semg0)
    s2, slog, sden, slast = compute(isl0, own0, rec00, rec01, carry)

    accout[0] = s2
    accout[1] = slog
    accout[2] = sden.astype(jnp.float32)
    accout[3] = slast
    accout[4] = nlog
    accout[5] = ncnt
    pltpu.sync_copy(accout, out.at[wid])


_mesh = plsc.VectorSubcoreMesh(
    core_axis_name="c", subcore_axis_name="s", num_cores=_NC,
    num_subcores=_NS)

_sc_call = pl.kernel(
    _body,
    out_type=jax.ShapeDtypeStruct((_NW, 6, 16), jnp.float32),
    mesh=_mesh,
    scratch_types=(
        [pltpu.VMEM((_B, 2), jnp.int32) for _ in range(2)]
        + [pltpu.VMEM((_NSUB, _SUB), jnp.int32) for _ in range(4)]
        + [pltpu.VMEM((_B, _ROW), jnp.float32) for _ in range(2)]
        + [pltpu.VMEM((_B, _ROW), jnp.float32) for _ in range(4)]
        + [
            pltpu.VMEM((_MROWS, 16), jnp.int32),
            pltpu.VMEM((_NPT, _ROW), jnp.float32),
            pltpu.VMEM((6, 16), jnp.float32),
        ]
        + [pltpu.SemaphoreType.DMA for _ in range(4)]),
    compiler_params=pltpu.CompilerParams(
        use_tc_tiling_on_sc=False, needs_layout_passes=False),
)


def kernel(poss_node, poss_edge, groundTruth, mask, edges):
    gt32 = groundTruth.astype(jnp.int32)
    meta8 = gt32 + 8 * mask.astype(jnp.int32)
    metaw = jnp.pad(meta8, (0, _NPAD - _N)).reshape(_NPAD // 4, 4)
    metaw = (metaw[:, 0] | (metaw[:, 1] << 8) | (metaw[:, 2] << 16)
             | (metaw[:, 3] << 24)).reshape(_MROWS, 16)
    pnode = jnp.pad(poss_node, ((0, _NPAD - _N), (0, _ROW - _C)),
                    constant_values=1.0)

    parts = _sc_call(poss_edge, edges, pnode, metaw)
    s2 = jnp.sum(parts[:, 0, :])
    slog = jnp.sum(parts[:, 1, :])
    den = jnp.sum(parts[:, 2, :])
    slast = jnp.sum(parts[:, 3, :])
    nlog = jnp.sum(parts[:, 4, :])
    ncnt = jnp.sum(parts[:, 5, :])

    loss = -nlog / ncnt
    semi = jnp.float32(_SEMI_LAMBDA) * (jnp.float32(_E) - slast) * s2
    el = -slog * jnp.float32(_EDGE_LAMBDA) / den
    el = el * jnp.float32(_EDGE_LAMBDA) / den
    return loss + semi + el


# speedup vs baseline: 1.1821x; 1.1821x over previous
import functools

import jax
import jax.numpy as jnp
from jax import lax
from jax.experimental import pallas as pl
from jax.experimental.pallas import tpu as pltpu
from jax.experimental.pallas import tpu_sc as plsc

_N = 100000
_E = 3200000
_C = 7
_ROW = 8
_SEMI_LAMBDA = 0.001
_EDGE_LAMBDA = 1.0

_NC = 2
_NS = 16
_NW = _NC * _NS
_EPW = _E // _NW
_B = 800
_NCHUNK = _EPW // _B
_NPAIR = (_NCHUNK - 1) // 2
_SUB = 80
_NSUB = _B // _SUB
_STEPS = _B // 16
_SROWS = 10
_NPT = 3136
_NPAD = _NW * _NPT
_NSTEPS = _NPT // 16
_MROWS = _NPAD // 64

_LN2 = 0.6931471805599453


def _vlog(x):
    bits = lax.bitcast_convert_type(x, jnp.int32)
    e = (bits >> 23) - 127
    m = lax.bitcast_convert_type(
        (bits & jnp.int32(0x007FFFFF)) | jnp.int32(0x3F800000), jnp.float32)
    big = m > jnp.float32(1.41421356)
    m = jnp.where(big, m * jnp.float32(0.5), m)
    e = jnp.where(big, e + 1, e)
    f = m - jnp.float32(1.0)
    z = f * f
    p = jnp.float32(1.4249322787e-1)
    p = p * f + jnp.float32(-1.6668057665e-1)
    p = p * f + jnp.float32(2.0000714765e-1)
    p = p * f + jnp.float32(-2.4999993993e-1)
    p = p * f + jnp.float32(3.3333331174e-1)
    r = f * z * p - jnp.float32(0.5) * z + f
    return r + e.astype(jnp.float32) * jnp.float32(_LN2)


def _body(*refs):
    (pe, edg, pnd, mta, out) = refs[:5]
    (isl0, isl1, sep00, sep01, sep10, sep11, own0, own1,
     rec00, rec01, rec10, rec11,
     metat, npn, accout,
     semi0, semi1, semg0, semg1) = refs[5:]

    wid = lax.axis_index("s") * _NC + lax.axis_index("c")
    la = lax.iota(jnp.int32, 16)
    cols = [jnp.full((16,), c, jnp.int32) for c in range(_ROW)]
    czero = jnp.zeros((16,), jnp.int32)
    cone = jnp.full((16,), 1, jnp.int32)
    zero = jnp.zeros((16,), jnp.float32)
    one = jnp.full((16,), 1.0, jnp.float32)

    def meta_of(i):
        w = plsc.load_gather(metat, [i >> 6, (i >> 2) & 15])
        return (w >> ((i & 3) << 3)) & 15

    ebase = wid * _EPW

    def pair_idx(islab, j):
        jj = j + j
        row = jj >> 4
        col = jj & 15
        i0 = plsc.load_gather(islab, [row, col])
        i1 = plsc.load_gather(islab, [row, col + 1])
        return i0, i1

    def start_fetch(ci, islab, ownbuf, sem):
        base = ebase + ci * _B
        pltpu.async_copy(edg.at[pl.ds(base // 8, _B // 8)], islab, sem)
        pltpu.async_copy(pe.at[pl.ds(base, _B)], ownbuf, sem)

    def wait_fetch(islab, ownbuf, sem):
        pltpu.make_async_copy(edg.at[pl.ds(0, _B // 8)], islab, sem).wait()
        pltpu.make_async_copy(pe.at[pl.ds(0, _B)], ownbuf, sem).wait()

    def separate(islab, sep0, sep1):
        def srow(r, _):
            for u in range(5):
                j = r * 80 + u * 16 + la
                i0, i1 = pair_idx(islab, j)
                sep0[r, pl.ds(u * 16, 16)] = i0
                sep1[r, pl.ds(u * 16, 16)] = i1
            return 0
        lax.fori_loop(0, _SROWS, srow, 0)

    def issue_gathers(sep0, sep1, r0buf, r1buf, sem):
        for r in range(_NSUB):
            pltpu.async_copy(
                pe.at[sep0.at[r]], r0buf.at[pl.ds(r * _SUB, _SUB)], sem)
            pltpu.async_copy(
                pe.at[sep1.at[r]], r1buf.at[pl.ds(r * _SUB, _SUB)], sem)

    def wait_gathers(r0buf, r1buf, sem):
        pltpu.make_async_copy(pe.at[pl.ds(0, _B)], r0buf, sem).wait()
        pltpu.make_async_copy(pe.at[pl.ds(0, _B)], r1buf, sem).wait()

    def compute(islab, own, rec0, rec1, carry):
        def step(t, c2):
            s2, slog, sden, slast = c2
            j = t * 16 + la
            i0, i1 = pair_idx(islab, j)
            acc = jnp.zeros((16,), jnp.float32)
            for c in range(_ROW):
                a = plsc.load_gather(rec0, [j, cols[c]])
                b = plsc.load_gather(rec1, [j, cols[c]])
                d = a - b
                acc = acc + d * d
            mt0 = meta_of(i0)
            mt1 = meta_of(i1)
            g0 = mt0 & 7
            g1 = mt1 & 7
            m0 = mt0 >> 3
            m1 = mt1 >> 3
            plast = plsc.load_gather(own, [j, cols[7]])
            p0 = plsc.load_gather(own, [j, g0])
            p1 = plsc.load_gather(own, [j, g1])
            m0b = m0 == 1
            m1b = m1 == 1
            bothb = (m0 & m1) == 1
            anyi = m0 | m1
            sameb = g0 == g1
            arg1 = jnp.where(sameb, p0, plast)
            arg2 = jnp.where(m0b, plast + p0,
                             jnp.where(m1b, plast + p1, one))
            arg = jnp.where(bothb, arg1, arg2)
            return (s2 + acc, slog + _vlog(arg), sden + anyi,
                    slast + plast)

        return lax.fori_loop(0, _STEPS, step, carry)

    start_fetch(0, isl0, own0, semi0)

    pltpu.sync_copy(mta, metat)
    nbase = wid * _NPT
    pltpu.sync_copy(pnd.at[pl.ds(nbase, _NPT)], npn)

    def nstep(t, carry):
        nlog, ncnt = carry
        j = t * 16 + la
        mt = meta_of(nbase + j)
        g = mt & 7
        mv = (mt >> 3).astype(jnp.float32)
        pn = plsc.load_gather(npn, [j, g])
        return nlog + mv * _vlog(pn), ncnt + mv

    nlog, ncnt = lax.fori_loop(0, _NSTEPS, nstep, (zero, zero))

    wait_fetch(isl0, own0, semi0)
    separate(isl0, sep00, sep01)
    issue_gathers(sep00, sep01, rec00, rec01, semg0)
    start_fetch(1, isl1, own1, semi1)

    izero = jnp.zeros((16,), jnp.int32)

    def pair(k, carry):
        c0 = 2 * k
        s2, slog, sden, slast = carry
        wait_fetch(isl1, own1, semi1)
        separate(isl1, sep10, sep11)
        issue_gathers(sep10, sep11, rec10, rec11, semg1)
        wait_gathers(rec00, rec01, semg0)
        carry = compute(isl0, own0, rec00, rec01, (s2, slog, sden, slast))
        start_fetch(c0 + 2, isl0, own0, semi0)
        wait_fetch(isl0, own0, semi0)
        separate(isl0, sep00, sep01)
        issue_gathers(sep00, sep01, rec00, rec01, semg0)
        wait_gathers(rec10, rec11, semg1)
        carry = compute(isl1, own1, rec10, rec11, carry)

        @pl.when(k < _NPAIR - 1)
        def _():
            start_fetch(c0 + 3, isl1, own1, semi1)
        return carry

    sdeni = izero
    carry = lax.fori_loop(0, _NPAIR, pair, (zero, zero, sdeni, zero))

    wait_gathers(rec00, rec01, semg0)
    s2, slog, sden, slast = compute(isl0, own0, rec00, rec01, carry)

    accout[0] = s2
    accout[1] = slog
    accout[2] = sden.astype(jnp.float32)
    accout[3] = slast
    accout[4] = nlog
    accout[5] = ncnt
    pltpu.sync_copy(accout, out.at[wid])


_mesh = plsc.VectorSubcoreMesh(
    core_axis_name="c", subcore_axis_name="s", num_cores=_NC,
    num_subcores=_NS)

_sc_call = pl.kernel(
    _body,
    out_type=jax.ShapeDtypeStruct((_NW, 6, 16), jnp.float32),
    mesh=_mesh,
    scratch_types=(
        [pltpu.VMEM((_B // 8, 16), jnp.int32) for _ in range(2)]
        + [pltpu.VMEM((_NSUB, _SUB), jnp.int32) for _ in range(4)]
        + [pltpu.VMEM((_B, _ROW), jnp.float32) for _ in range(2)]
        + [pltpu.VMEM((_B, _ROW), jnp.float32) for _ in range(4)]
        + [
            pltpu.VMEM((_MROWS, 16), jnp.int32),
            pltpu.VMEM((_NPT, _ROW), jnp.float32),
            pltpu.VMEM((6, 16), jnp.float32),
        ]
        + [pltpu.SemaphoreType.DMA for _ in range(4)]),
    compiler_params=pltpu.CompilerParams(
        use_tc_tiling_on_sc=False, needs_layout_passes=False),
)


def kernel(poss_node, poss_edge, groundTruth, mask, edges):
    gt32 = groundTruth.astype(jnp.int32)
    meta8 = gt32 + 8 * mask.astype(jnp.int32)
    metaw = jnp.pad(meta8, (0, _NPAD - _N)).reshape(_NPAD // 4, 4)
    metaw = (metaw[:, 0] | (metaw[:, 1] << 8) | (metaw[:, 2] << 16)
             | (metaw[:, 3] << 24)).reshape(_MROWS, 16)
    pnode = jnp.pad(poss_node, ((0, _NPAD - _N), (0, _ROW - _C)),
                    constant_values=1.0)

    parts = _sc_call(poss_edge, edges.reshape(2 * _E // 16, 16), pnode, metaw)
    s2 = jnp.sum(parts[:, 0, :])
    slog = jnp.sum(parts[:, 1, :])
    den = jnp.sum(parts[:, 2, :])
    slast = jnp.sum(parts[:, 3, :])
    nlog = jnp.sum(parts[:, 4, :])
    ncnt = jnp.sum(parts[:, 5, :])

    loss = -nlog / ncnt
    semi = jnp.float32(_SEMI_LAMBDA) * (jnp.float32(_E) - slast) * s2
    el = -slog * jnp.float32(_EDGE_LAMBDA) / den
    el = el * jnp.float32(_EDGE_LAMBDA) / den
    return loss + semi + el


# speedup vs baseline: 4.0614x; 3.4358x over previous
import functools

import jax
import jax.numpy as jnp
from jax import lax
from jax.experimental import pallas as pl
from jax.experimental.pallas import tpu as pltpu
from jax.experimental.pallas import tpu_sc as plsc

_N = 100000
_E = 3200000
_C = 7
_ROW = 8
_SEMI_LAMBDA = 0.001
_EDGE_LAMBDA = 1.0

_NC = 2
_NS = 16
_NW = _NC * _NS
_EPW = _E // _NW
_B = 800
_NCHUNK = _EPW // _B
_NPAIR = (_NCHUNK - 1) // 2
_SUB = 80
_NSUB = _B // _SUB
_STEPS = _B // 16
_SROWS = 10
_NPT = 3136
_NPAD = _NW * _NPT
_NSTEPS = _NPT // 16
_MROWS = _NPAD // 64

_LN2 = 0.6931471805599453


def _vlog(x):
    bits = lax.bitcast_convert_type(x, jnp.int32)
    e = (bits >> 23) - 127
    m = lax.bitcast_convert_type(
        (bits & jnp.int32(0x007FFFFF)) | jnp.int32(0x3F800000), jnp.float32)
    big = m > jnp.float32(1.41421356)
    m = jnp.where(big, m * jnp.float32(0.5), m)
    e = jnp.where(big, e + 1, e)
    f = m - jnp.float32(1.0)
    z = f * f
    p = jnp.float32(1.4249322787e-1)
    p = p * f + jnp.float32(-1.6668057665e-1)
    p = p * f + jnp.float32(2.0000714765e-1)
    p = p * f + jnp.float32(-2.4999993993e-1)
    p = p * f + jnp.float32(3.3333331174e-1)
    r = f * z * p - jnp.float32(0.5) * z + f
    return r + e.astype(jnp.float32) * jnp.float32(_LN2)


def _body(*refs):
    (pe, edg, pnd, mta, out) = refs[:5]
    (isl00, isl01, isl10, isl11, own0, own1,
     rec00, rec01, rec10, rec11,
     metat, npn, accout,
     semi0, semi1, semg0, semg1) = refs[5:]

    wid = lax.axis_index("s") * _NC + lax.axis_index("c")
    la = lax.iota(jnp.int32, 16)
    cols = [jnp.full((16,), c, jnp.int32) for c in range(_ROW)]
    czero = jnp.zeros((16,), jnp.int32)
    cone = jnp.full((16,), 1, jnp.int32)
    zero = jnp.zeros((16,), jnp.float32)
    one = jnp.full((16,), 1.0, jnp.float32)

    def meta_of(i):
        w = plsc.load_gather(metat, [i >> 6, (i >> 2) & 15])
        return (w >> ((i & 3) << 3)) & 15

    ebase = wid * _EPW

    def start_fetch(ci, sl0, sl1, ownbuf, sem):
        base = ebase + ci * _B
        pltpu.async_copy(edg.at[0, pl.ds(base, _B)], sl0, sem)
        pltpu.async_copy(edg.at[1, pl.ds(base, _B)], sl1, sem)
        pltpu.async_copy(pe.at[pl.ds(base, _B)], ownbuf, sem)

    def wait_fetch(sl0, sl1, ownbuf, sem):
        pltpu.make_async_copy(edg.at[0, pl.ds(0, _B)], sl0, sem).wait()
        pltpu.make_async_copy(edg.at[0, pl.ds(0, _B)], sl1, sem).wait()
        pltpu.make_async_copy(pe.at[pl.ds(0, _B)], ownbuf, sem).wait()

    def issue_gathers(sl0, sl1, r0buf, r1buf, sem):
        for r in range(_NSUB):
            pltpu.async_copy(
                pe.at[sl0.at[pl.ds(r * _SUB, _SUB)]],
                r0buf.at[pl.ds(r * _SUB, _SUB)], sem)
            pltpu.async_copy(
                pe.at[sl1.at[pl.ds(r * _SUB, _SUB)]],
                r1buf.at[pl.ds(r * _SUB, _SUB)], sem)

    def wait_gathers(r0buf, r1buf, sem):
        pltpu.make_async_copy(pe.at[pl.ds(0, _B)], r0buf, sem).wait()
        pltpu.make_async_copy(pe.at[pl.ds(0, _B)], r1buf, sem).wait()

    def compute(sl0, sl1, own, rec0, rec1, carry):
        def step(t, c2):
            s2, slog, sden, slast = c2
            j = t * 16 + la
            i0 = sl0[pl.ds(t * 16, 16)]
            i1 = sl1[pl.ds(t * 16, 16)]
            acc = jnp.zeros((16,), jnp.float32)
            for c in range(_ROW):
                a = plsc.load_gather(rec0, [j, cols[c]])
                b = plsc.load_gather(rec1, [j, cols[c]])
                d = a - b
                acc = acc + d * d
            mt0 = meta_of(i0)
            mt1 = meta_of(i1)
            g0 = mt0 & 7
            g1 = mt1 & 7
            m0 = mt0 >> 3
            m1 = mt1 >> 3
            plast = plsc.load_gather(own, [j, cols[7]])
            p0 = plsc.load_gather(own, [j, g0])
            p1 = plsc.load_gather(own, [j, g1])
            m0b = m0 == 1
            m1b = m1 == 1
            bothb = (m0 & m1) == 1
            anyi = m0 | m1
            sameb = g0 == g1
            arg1 = jnp.where(sameb, p0, plast)
            arg2 = jnp.where(m0b, plast + p0,
                             jnp.where(m1b, plast + p1, one))
            arg = jnp.where(bothb, arg1, arg2)
            return (s2 + acc, slog + _vlog(arg), sden + anyi,
                    slast + plast)

        return lax.fori_loop(0, _STEPS, step, carry)

    start_fetch(0, isl00, isl01, own0, semi0)

    pltpu.sync_copy(mta, metat)
    nbase = wid * _NPT
    pltpu.sync_copy(pnd.at[pl.ds(nbase, _NPT)], npn)

    def nstep(t, carry):
        nlog, ncnt = carry
        j = t * 16 + la
        mt = meta_of(nbase + j)
        g = mt & 7
        mv = (mt >> 3).astype(jnp.float32)
        pn = plsc.load_gather(npn, [j, g])
        return nlog + mv * _vlog(pn), ncnt + mv

    nlog, ncnt = lax.fori_loop(0, _NSTEPS, nstep, (zero, zero))

    wait_fetch(isl00, isl01, own0, semi0)
    issue_gathers(isl00, isl01, rec00, rec01, semg0)
    start_fetch(1, isl10, isl11, own1, semi1)

    izero = jnp.zeros((16,), jnp.int32)

    def pair(k, carry):
        c0 = 2 * k
        s2, slog, sden, slast = carry
        wait_fetch(isl10, isl11, own1, semi1)
        issue_gathers(isl10, isl11, rec10, rec11, semg1)
        wait_gathers(rec00, rec01, semg0)
        carry = compute(isl00, isl01, own0, rec00, rec01,
                        (s2, slog, sden, slast))
        start_fetch(c0 + 2, isl00, isl01, own0, semi0)
        wait_fetch(isl00, isl01, own0, semi0)
        issue_gathers(isl00, isl01, rec00, rec01, semg0)
        wait_gathers(rec10, rec11, semg1)
        carry = compute(isl10, isl11, own1, rec10, rec11, carry)

        @pl.when(k < _NPAIR - 1)
        def _():
            start_fetch(c0 + 3, isl10, isl11, own1, semi1)
        return carry

    sdeni = izero
    carry = lax.fori_loop(0, _NPAIR, pair, (zero, zero, sdeni, zero))

    wait_gathers(rec00, rec01, semg0)
    s2, slog, sden, slast = compute(isl00, isl01, own0, rec00, rec01, carry)

    accout[0] = s2
    accout[1] = slog
    accout[2] = sden.astype(jnp.float32)
    accout[3] = slast
    accout[4] = nlog
    accout[5] = ncnt
    pltpu.sync_copy(accout, out.at[wid])


_mesh = plsc.VectorSubcoreMesh(
    core_axis_name="c", subcore_axis_name="s", num_cores=_NC,
    num_subcores=_NS)

_sc_call = pl.kernel(
    _body,
    out_type=jax.ShapeDtypeStruct((_NW, 6, 16), jnp.float32),
    mesh=_mesh,
    scratch_types=(
        [pltpu.VMEM((_B,), jnp.int32) for _ in range(4)]
        + [pltpu.VMEM((_B, _ROW), jnp.float32) for _ in range(2)]
        + [pltpu.VMEM((_B, _ROW), jnp.float32) for _ in range(4)]
        + [
            pltpu.VMEM((_MROWS, 16), jnp.int32),
            pltpu.VMEM((_NPT, _ROW), jnp.float32),
            pltpu.VMEM((6, 16), jnp.float32),
        ]
        + [pltpu.SemaphoreType.DMA for _ in range(4)]),
    compiler_params=pltpu.CompilerParams(
        use_tc_tiling_on_sc=False, needs_layout_passes=False),
)


def kernel(poss_node, poss_edge, groundTruth, mask, edges):
    gt32 = groundTruth.astype(jnp.int32)
    meta8 = gt32 + 8 * mask.astype(jnp.int32)
    metaw = jnp.pad(meta8, (0, _NPAD - _N)).reshape(_NPAD // 4, 4)
    metaw = (metaw[:, 0] | (metaw[:, 1] << 8) | (metaw[:, 2] << 16)
             | (metaw[:, 3] << 24)).reshape(_MROWS, 16)
    pnode = jnp.pad(poss_node, ((0, _NPAD - _N), (0, _ROW - _C)),
                    constant_values=1.0)

    parts = _sc_call(poss_edge, edges.T, pnode, metaw)
    s2 = jnp.sum(parts[:, 0, :])
    slog = jnp.sum(parts[:, 1, :])
    den = jnp.sum(parts[:, 2, :])
    slast = jnp.sum(parts[:, 3, :])
    nlog = jnp.sum(parts[:, 4, :])
    ncnt = jnp.sum(parts[:, 5, :])

    loss = -nlog / ncnt
    semi = jnp.float32(_SEMI_LAMBDA) * (jnp.float32(_E) - slast) * s2
    el = -slog * jnp.float32(_EDGE_LAMBDA) / den
    el = el * jnp.float32(_EDGE_LAMBDA) / den
    return loss + semi + el
